# Initial kernel scaffold; baseline (speedup 1.0000x reference)
import jax
import jax.numpy as jnp
from jax.experimental import pallas as pl


def kernel(x, edge_index, W1, b1, W2, b2):
    raise NotImplementedError("write your pallas kernel here")


# speedup vs baseline: 20.0221x; 20.0221x over previous
import functools

import jax
import jax.numpy as jnp
from jax import lax
from jax.experimental import pallas as pl
from jax.experimental.pallas import tpu as pltpu
from jax.experimental.pallas import tpu_sc as plsc

N = 10000
E = 320000
D = 128
NC = 2
NS = 16
NW = NC * NS
B = 80
ROWS = E // B
RPT = ROWS // NW
NP = 10240
TR = NP // NS


def _mesh():
    return plsc.VectorSubcoreMesh(core_axis_name="c", subcore_axis_name="s")


@functools.partial(
    pl.kernel,
    out_type=jax.ShapeDtypeStruct((NC, NP), jnp.float32),
    mesh=_mesh(),
    scratch_types=[
        pltpu.VMEM((RPT, B), jnp.int32),
        pltpu.VMEM((B,), jnp.float32),
        pltpu.VMEM((TR,), jnp.float32),
        pltpu.VMEM_SHARED((NP,), jnp.float32),
    ],
)
def _deg_kernel(dst_hbm, out_hbm, idx_v, ones_v, zbuf_v, acc_sh):
    c = lax.axis_index("c")
    s = lax.axis_index("s")
    wid = s * NC + c

    def fill_ones(i, _):
        ones_v[pl.ds(i * 16, 16)] = jnp.full((16,), 1.0, jnp.float32)
        return 0

    lax.fori_loop(0, B // 16, fill_ones, 0)

    def fill_zero(i, _):
        zbuf_v[pl.ds(i * 16, 16)] = jnp.zeros((16,), jnp.float32)
        return 0

    lax.fori_loop(0, TR // 16, fill_zero, 0)

    pltpu.sync_copy(zbuf_v, acc_sh.at[pl.ds(s * TR, TR)])
    pltpu.sync_copy(dst_hbm.at[wid], idx_v)
    plsc.subcore_barrier()

    def body(j, _):
        pltpu.sync_copy(ones_v, acc_sh.at[idx_v.at[j]], add=True)
        return 0

    lax.fori_loop(0, RPT, body, 0)
    plsc.subcore_barrier()
    pltpu.sync_copy(acc_sh.at[pl.ds(s * TR, TR)],
                    out_hbm.at[c, pl.ds(s * TR, TR)])


@functools.partial(
    pl.kernel,
    out_type=jax.ShapeDtypeStruct((NC, NP, D), jnp.float32),
    mesh=_mesh(),
    scratch_types=[
        pltpu.VMEM((RPT, B), jnp.int32),
        pltpu.VMEM((RPT, B), jnp.int32),
        pltpu.VMEM((B, D), jnp.float32),
        pltpu.VMEM_SHARED((NP, D), jnp.float32),
        pltpu.SemaphoreType.DMA,
    ],
)
def _edge_kernel(y_hbm, src_hbm, dst_hbm, out_hbm,
                 src_v, dst_v, rows_v, acc_sh, sem):
    c = lax.axis_index("c")
    s = lax.axis_index("s")
    wid = s * NC + c

    def zrow(r, _):
        def zcol(k, _):
            rows_v[r, pl.ds(k * 16, 16)] = jnp.zeros((16,), jnp.float32)
            return 0

        lax.fori_loop(0, D // 16, zcol, 0)
        return 0

    lax.fori_loop(0, B, zrow, 0)

    for u in range(TR // B):
        pltpu.sync_copy(rows_v, acc_sh.at[pl.ds(s * TR + u * B, B)])

    pltpu.sync_copy(src_hbm.at[wid], src_v)
    pltpu.sync_copy(dst_hbm.at[wid], dst_v)
    plsc.subcore_barrier()

    def body(j, _):
        pltpu.async_copy(y_hbm.at[src_v.at[j]], rows_v, sem).wait()
        pltpu.sync_copy(rows_v, acc_sh.at[dst_v.at[j]], add=True)
        return 0

    lax.fori_loop(0, RPT, body, 0)
    plsc.subcore_barrier()
    pltpu.sync_copy(acc_sh.at[pl.ds(s * TR, TR)],
                    out_hbm.at[c, pl.ds(s * TR, TR)])


_RB = 1000
_G = N // _RB


def _rows_spec():
    return pl.BlockSpec((_RB, D), lambda i: (i, 0))


def _matmul_scale(x, w, dis):
    def body(x_ref, w_ref, d_ref, y_ref):
        y_ref[...] = jnp.dot(x_ref[...], w_ref[...],
                             preferred_element_type=jnp.float32) * d_ref[...]

    return pl.pallas_call(
        body,
        grid=(_G,),
        in_specs=[_rows_spec(),
                  pl.BlockSpec((D, D), lambda i: (0, 0)),
                  pl.BlockSpec((_RB, 1), lambda i: (i, 0))],
        out_specs=_rows_spec(),
        out_shape=jax.ShapeDtypeStruct((N, D), jnp.float32),
    )(x, w, dis)


def _mid_layer(s0, s1, y1, dis, b1, w2):
    def body(s0_ref, s1_ref, y_ref, d_ref, b_ref, w_ref, o_ref):
        d = d_ref[...]
        h = jnp.maximum(d * (s0_ref[...] + s1_ref[...] + y_ref[...])
                        + b_ref[...], 0.0)
        o_ref[...] = jnp.dot(h, w_ref[...],
                             preferred_element_type=jnp.float32) * d

    return pl.pallas_call(
        body,
        grid=(_G,),
        in_specs=[_rows_spec(), _rows_spec(), _rows_spec(),
                  pl.BlockSpec((_RB, 1), lambda i: (i, 0)),
                  pl.BlockSpec((1, D), lambda i: (0, 0)),
                  pl.BlockSpec((D, D), lambda i: (0, 0))],
        out_specs=_rows_spec(),
        out_shape=jax.ShapeDtypeStruct((N, D), jnp.float32),
    )(s0, s1, y1, dis, b1, w2)


def _final_layer(s0, s1, y2, dis, b2):
    def body(s0_ref, s1_ref, y_ref, d_ref, b_ref, o_ref):
        o_ref[...] = (d_ref[...] * (s0_ref[...] + s1_ref[...] + y_ref[...])
                      + b_ref[...])

    return pl.pallas_call(
        body,
        grid=(_G,),
        in_specs=[_rows_spec(), _rows_spec(), _rows_spec(),
                  pl.BlockSpec((_RB, 1), lambda i: (i, 0)),
                  pl.BlockSpec((1, D), lambda i: (0, 0))],
        out_specs=_rows_spec(),
        out_shape=jax.ShapeDtypeStruct((N, D), jnp.float32),
    )(s0, s1, y2, dis, b2)


def kernel(x, edge_index, W1, b1, W2, b2):
    ei = edge_index.astype(jnp.int32)
    src2 = ei[0].reshape(NW, RPT, B)
    dst2 = ei[1].reshape(NW, RPT, B)

    dp = _deg_kernel(dst2)
    deg = dp[0, :N] + dp[1, :N] + 1.0
    dis = lax.rsqrt(deg)[:, None]

    y1 = _matmul_scale(x, W1, dis)
    sp1 = _edge_kernel(y1, src2, dst2)
    y2 = _mid_layer(sp1[0, :N], sp1[1, :N], y1, dis,
                    b1.reshape(1, D), W2)
    sp2 = _edge_kernel(y2, src2, dst2)
    return _final_layer(sp2[0, :N], sp2[1, :N], y2, dis,
                        b2.reshape(1, D))


# speedup vs baseline: 24.3211x; 1.2147x over previous
import functools

import jax
import jax.numpy as jnp
from jax import lax
from jax.experimental import pallas as pl
from jax.experimental.pallas import tpu as pltpu
from jax.experimental.pallas import tpu_sc as plsc

N = 10000
E = 320000
D = 128
NC = 2
NS = 16
NW = NC * NS
B = 80
ROWS = E // B
RPT = ROWS // NW
NP = 10240
TR = NP // NS
IB = 25
NB = RPT // IB


def _mesh():
    return plsc.VectorSubcoreMesh(core_axis_name="c", subcore_axis_name="s")


@functools.partial(
    pl.kernel,
    out_type=jax.ShapeDtypeStruct((NC, NP), jnp.float32),
    mesh=_mesh(),
    scratch_types=[
        pltpu.VMEM((RPT, B), jnp.int32),
        pltpu.VMEM((B,), jnp.float32),
        pltpu.VMEM((TR,), jnp.float32),
        pltpu.VMEM_SHARED((NP,), jnp.float32),
    ],
)
def _deg_kernel(dst_hbm, out_hbm, idx_v, ones_v, zbuf_v, acc_sh):
    c = lax.axis_index("c")
    s = lax.axis_index("s")
    wid = s * NC + c

    def fill_ones(i, _):
        ones_v[pl.ds(i * 16, 16)] = jnp.full((16,), 1.0, jnp.float32)
        return 0

    lax.fori_loop(0, B // 16, fill_ones, 0)

    def fill_zero(i, _):
        zbuf_v[pl.ds(i * 16, 16)] = jnp.zeros((16,), jnp.float32)
        return 0

    lax.fori_loop(0, TR // 16, fill_zero, 0)

    pltpu.sync_copy(zbuf_v, acc_sh.at[pl.ds(s * TR, TR)])
    pltpu.sync_copy(dst_hbm.at[wid], idx_v)
    plsc.subcore_barrier()

    def body(j, _):
        pltpu.sync_copy(ones_v, acc_sh.at[idx_v.at[j]], add=True)
        return 0

    lax.fori_loop(0, RPT, body, 0)
    plsc.subcore_barrier()
    pltpu.sync_copy(acc_sh.at[pl.ds(s * TR, TR)],
                    out_hbm.at[c, pl.ds(s * TR, TR)])


@functools.partial(
    pl.kernel,
    out_type=jax.ShapeDtypeStruct((NC, NP, D), jnp.float32),
    mesh=_mesh(),
    scratch_types=[
        pltpu.VMEM((IB, B), jnp.int32),
        pltpu.VMEM((IB, B), jnp.int32),
        pltpu.VMEM((2, B, D), jnp.float32),
        pltpu.VMEM_SHARED((NP, D), jnp.float32),
        pltpu.SemaphoreType.DMA,
    ],
)
def _edge_kernel(y_hbm, src_hbm, dst_hbm, out_hbm,
                 src_v, dst_v, rows_v, acc_sh, sem):
    c = lax.axis_index("c")
    s = lax.axis_index("s")
    wid = s * NC + c

    def zrow(r, _):
        def zcol(k, _):
            rows_v[0, r, pl.ds(k * 16, 16)] = jnp.zeros((16,), jnp.float32)
            return 0

        lax.fori_loop(0, D // 16, zcol, 0)
        return 0

    lax.fori_loop(0, B, zrow, 0)

    for u in range(TR // B):
        pltpu.sync_copy(rows_v.at[0], acc_sh.at[pl.ds(s * TR + u * B, B)])

    plsc.subcore_barrier()

    def block(bk, _):
        pltpu.sync_copy(src_hbm.at[wid, bk], src_v)
        pltpu.sync_copy(dst_hbm.at[wid, bk], dst_v)
        pltpu.async_copy(y_hbm.at[src_v.at[0]], rows_v.at[0], sem)

        def body(j, _):
            cur = lax.rem(j, 2)
            pltpu.make_async_copy(y_hbm.at[src_v.at[j]], rows_v.at[cur],
                                  sem).wait()

            @pl.when(j + 1 < IB)
            def _():
                pltpu.async_copy(y_hbm.at[src_v.at[j + 1]],
                                 rows_v.at[1 - cur], sem)

            pltpu.sync_copy(rows_v.at[cur], acc_sh.at[dst_v.at[j]], add=True)
            return 0

        lax.fori_loop(0, IB, body, 0)
        return 0

    lax.fori_loop(0, NB, block, 0)
    plsc.subcore_barrier()
    pltpu.sync_copy(acc_sh.at[pl.ds(s * TR, TR)],
                    out_hbm.at[c, pl.ds(s * TR, TR)])


_RB = 1000
_G = N // _RB


def _rows_spec():
    return pl.BlockSpec((_RB, D), lambda i: (i, 0))


def _matmul_scale(x, w, dis):
    def body(x_ref, w_ref, d_ref, y_ref):
        y_ref[...] = jnp.dot(x_ref[...], w_ref[...],
                             preferred_element_type=jnp.float32) * d_ref[...]

    return pl.pallas_call(
        body,
        grid=(_G,),
        in_specs=[_rows_spec(),
                  pl.BlockSpec((D, D), lambda i: (0, 0)),
                  pl.BlockSpec((_RB, 1), lambda i: (i, 0))],
        out_specs=_rows_spec(),
        out_shape=jax.ShapeDtypeStruct((N, D), jnp.float32),
    )(x, w, dis)


def _mid_layer(s0, s1, y1, dis, b1, w2):
    def body(s0_ref, s1_ref, y_ref, d_ref, b_ref, w_ref, o_ref):
        d = d_ref[...]
        h = jnp.maximum(d * (s0_ref[...] + s1_ref[...] + y_ref[...])
                        + b_ref[...], 0.0)
        o_ref[...] = jnp.dot(h, w_ref[...],
                             preferred_element_type=jnp.float32) * d

    return pl.pallas_call(
        body,
        grid=(_G,),
        in_specs=[_rows_spec(), _rows_spec(), _rows_spec(),
                  pl.BlockSpec((_RB, 1), lambda i: (i, 0)),
                  pl.BlockSpec((1, D), lambda i: (0, 0)),
                  pl.BlockSpec((D, D), lambda i: (0, 0))],
        out_specs=_rows_spec(),
        out_shape=jax.ShapeDtypeStruct((N, D), jnp.float32),
    )(s0, s1, y1, dis, b1, w2)


def _final_layer(s0, s1, y2, dis, b2):
    def body(s0_ref, s1_ref, y_ref, d_ref, b_ref, o_ref):
        o_ref[...] = (d_ref[...] * (s0_ref[...] + s1_ref[...] + y_ref[...])
                      + b_ref[...])

    return pl.pallas_call(
        body,
        grid=(_G,),
        in_specs=[_rows_spec(), _rows_spec(), _rows_spec(),
                  pl.BlockSpec((_RB, 1), lambda i: (i, 0)),
                  pl.BlockSpec((1, D), lambda i: (0, 0))],
        out_specs=_rows_spec(),
        out_shape=jax.ShapeDtypeStruct((N, D), jnp.float32),
    )(s0, s1, y2, dis, b2)


def kernel(x, edge_index, W1, b1, W2, b2):
    ei = edge_index.astype(jnp.int32)
    src2 = ei[0].reshape(NW, NB, IB, B)
    dst2 = ei[1].reshape(NW, NB, IB, B)
    dstd = ei[1].reshape(NW, RPT, B)

    dp = _deg_kernel(dstd)
    deg = dp[0, :N] + dp[1, :N] + 1.0
    dis = lax.rsqrt(deg)[:, None]

    y1 = _matmul_scale(x, W1, dis)
    sp1 = _edge_kernel(y1, src2, dst2)
    y2 = _mid_layer(sp1[0, :N], sp1[1, :N], y1, dis,
                    b1.reshape(1, D), W2)
    sp2 = _edge_kernel(y2, src2, dst2)
    return _final_layer(sp2[0, :N], sp2[1, :N], y2, dis,
                        b2.reshape(1, D))


# speedup vs baseline: 34.4427x; 1.4162x over previous
import functools

import jax
import jax.numpy as jnp
from jax import lax
from jax.experimental import pallas as pl
from jax.experimental.pallas import tpu as pltpu
from jax.experimental.pallas import tpu_sc as plsc

N = 10000
E = 320000
D = 128
NC = 2
NS = 16
NW = NC * NS
B = 80
ROWS = E // B
RPT = ROWS // NW
NP = 10240
TR = NP // NS
IB = 25
NB = RPT // IB


def _mesh():
    return plsc.VectorSubcoreMesh(core_axis_name="c", subcore_axis_name="s")


@functools.partial(
    pl.kernel,
    out_type=jax.ShapeDtypeStruct((NC, NP), jnp.float32),
    mesh=_mesh(),
    scratch_types=[
        pltpu.VMEM((RPT, B), jnp.int32),
        pltpu.VMEM((B,), jnp.float32),
        pltpu.VMEM((TR,), jnp.float32),
        pltpu.VMEM_SHARED((NP,), jnp.float32),
    ],
)
def _deg_kernel(dst_hbm, out_hbm, idx_v, ones_v, zbuf_v, acc_sh):
    c = lax.axis_index("c")
    s = lax.axis_index("s")
    wid = s * NC + c

    def fill_ones(i, _):
        ones_v[pl.ds(i * 16, 16)] = jnp.full((16,), 1.0, jnp.float32)
        return 0

    lax.fori_loop(0, B // 16, fill_ones, 0)

    def fill_zero(i, _):
        zbuf_v[pl.ds(i * 16, 16)] = jnp.zeros((16,), jnp.float32)
        return 0

    lax.fori_loop(0, TR // 16, fill_zero, 0)

    pltpu.sync_copy(zbuf_v, acc_sh.at[pl.ds(s * TR, TR)])
    pltpu.sync_copy(dst_hbm.at[wid], idx_v)
    plsc.subcore_barrier()

    def body(j, _):
        pltpu.sync_copy(ones_v, acc_sh.at[idx_v.at[j]], add=True)
        return 0

    lax.fori_loop(0, RPT, body, 0)
    plsc.subcore_barrier()
    pltpu.sync_copy(acc_sh.at[pl.ds(s * TR, TR)],
                    out_hbm.at[c, pl.ds(s * TR, TR)])


@functools.partial(
    pl.kernel,
    out_type=jax.ShapeDtypeStruct((NC, NP, D), jnp.float32),
    mesh=_mesh(),
    scratch_types=[
        pltpu.VMEM((IB, B), jnp.int32),
        pltpu.VMEM((IB, B), jnp.int32),
        pltpu.VMEM((3, B, D), jnp.float32),
        pltpu.VMEM_SHARED((NP, D), jnp.float32),
        pltpu.SemaphoreType.DMA,
        pltpu.SemaphoreType.DMA,
    ],
)
def _edge_kernel(y_hbm, src_hbm, dst_hbm, out_hbm,
                 src_v, dst_v, rows_v, acc_sh, sem_g, sem_s):
    c = lax.axis_index("c")
    s = lax.axis_index("s")
    wid = s * NC + c

    def zrow(r, _):
        def zcol(k, _):
            rows_v[0, r, pl.ds(k * 16, 16)] = jnp.zeros((16,), jnp.float32)
            return 0

        lax.fori_loop(0, D // 16, zcol, 0)
        return 0

    lax.fori_loop(0, B, zrow, 0)

    for u in range(TR // B):
        pltpu.sync_copy(rows_v.at[0], acc_sh.at[pl.ds(s * TR + u * B, B)])

    plsc.subcore_barrier()

    def block(bk, _):
        pltpu.sync_copy(src_hbm.at[wid, bk], src_v)
        pltpu.sync_copy(dst_hbm.at[wid, bk], dst_v)
        pltpu.async_copy(y_hbm.at[src_v.at[0]], rows_v.at[0], sem_g)
        pltpu.async_copy(y_hbm.at[src_v.at[1]], rows_v.at[1], sem_g)

        def body(j, _):
            cur = lax.rem(j, 3)
            pltpu.make_async_copy(y_hbm.at[src_v.at[j]], rows_v.at[cur],
                                  sem_g).wait()

            @pl.when(j >= 1)
            def _():
                prev = lax.rem(j + 2, 3)
                pltpu.make_async_copy(rows_v.at[prev],
                                      acc_sh.at[dst_v.at[j - 1]],
                                      sem_s).wait()

            @pl.when(j + 2 < IB)
            def _():
                pltpu.async_copy(y_hbm.at[src_v.at[j + 2]],
                                 rows_v.at[lax.rem(j + 2, 3)], sem_g)

            pltpu.async_copy(rows_v.at[cur], acc_sh.at[dst_v.at[j]], sem_s,
                             add=True)
            return 0

        lax.fori_loop(0, IB, body, 0)
        pltpu.make_async_copy(rows_v.at[lax.rem(IB - 1, 3)],
                              acc_sh.at[dst_v.at[IB - 1]], sem_s).wait()
        return 0

    lax.fori_loop(0, NB, block, 0)
    plsc.subcore_barrier()
    pltpu.sync_copy(acc_sh.at[pl.ds(s * TR, TR)],
                    out_hbm.at[c, pl.ds(s * TR, TR)])


_RB = 1000
_G = N // _RB


def _rows_spec():
    return pl.BlockSpec((_RB, D), lambda i: (i, 0))


def _matmul_scale(x, w, dis):
    def body(x_ref, w_ref, d_ref, y_ref):
        y_ref[...] = jnp.dot(x_ref[...], w_ref[...],
                             preferred_element_type=jnp.float32) * d_ref[...]

    return pl.pallas_call(
        body,
        grid=(_G,),
        in_specs=[_rows_spec(),
                  pl.BlockSpec((D, D), lambda i: (0, 0)),
                  pl.BlockSpec((_RB, 1), lambda i: (i, 0))],
        out_specs=_rows_spec(),
        out_shape=jax.ShapeDtypeStruct((N, D), jnp.float32),
    )(x, w, dis)


def _part_spec(p):
    return pl.BlockSpec((1, _RB, D), lambda i, _p=p: (_p, i, 0))


def _mid_layer(sp, y1, dis, b1, w2):
    def body(s0_ref, s1_ref, y_ref, d_ref, b_ref, w_ref, o_ref):
        d = d_ref[...]
        h = jnp.maximum(d * (s0_ref[0] + s1_ref[0] + y_ref[...])
                        + b_ref[...], 0.0)
        o_ref[...] = jnp.dot(h, w_ref[...],
                             preferred_element_type=jnp.float32) * d

    return pl.pallas_call(
        body,
        grid=(_G,),
        in_specs=[_part_spec(0), _part_spec(1), _rows_spec(),
                  pl.BlockSpec((_RB, 1), lambda i: (i, 0)),
                  pl.BlockSpec((1, D), lambda i: (0, 0)),
                  pl.BlockSpec((D, D), lambda i: (0, 0))],
        out_specs=_rows_spec(),
        out_shape=jax.ShapeDtypeStruct((N, D), jnp.float32),
    )(sp, sp, y1, dis, b1, w2)


def _final_layer(sp, y2, dis, b2):
    def body(s0_ref, s1_ref, y_ref, d_ref, b_ref, o_ref):
        o_ref[...] = (d_ref[...] * (s0_ref[0] + s1_ref[0] + y_ref[...])
                      + b_ref[...])

    return pl.pallas_call(
        body,
        grid=(_G,),
        in_specs=[_part_spec(0), _part_spec(1), _rows_spec(),
                  pl.BlockSpec((_RB, 1), lambda i: (i, 0)),
                  pl.BlockSpec((1, D), lambda i: (0, 0))],
        out_specs=_rows_spec(),
        out_shape=jax.ShapeDtypeStruct((N, D), jnp.float32),
    )(sp, sp, y2, dis, b2)


def kernel(x, edge_index, W1, b1, W2, b2):
    ei = edge_index.astype(jnp.int32)
    src2 = ei[0].reshape(NW, NB, IB, B)
    dst2 = ei[1].reshape(NW, NB, IB, B)
    dstd = ei[1].reshape(NW, RPT, B)

    dp = _deg_kernel(dstd)
    deg = dp[0, :N] + dp[1, :N] + 1.0
    dis = lax.rsqrt(deg)[:, None]

    y1 = _matmul_scale(x, W1, dis)
    sp1 = _edge_kernel(y1, src2, dst2)
    y2 = _mid_layer(sp1, y1, dis, b1.reshape(1, D), W2)
    sp2 = _edge_kernel(y2, src2, dst2)
    return _final_layer(sp2, y2, dis, b2.reshape(1, D))


# speedup vs baseline: 36.3353x; 1.0549x over previous
import functools

import jax
import jax.numpy as jnp
from jax import lax
from jax.experimental import pallas as pl
from jax.experimental.pallas import tpu as pltpu
from jax.experimental.pallas import tpu_sc as plsc

N = 10000
E = 320000
D = 128
NC = 2
NS = 16
NW = NC * NS
B = 100
ROWS = E // B
RPT = ROWS // NW
NP = 10240
TR = NP // NS
IB = 25
NB = RPT // IB


def _mesh():
    return plsc.VectorSubcoreMesh(core_axis_name="c", subcore_axis_name="s")


@functools.partial(
    pl.kernel,
    out_type=jax.ShapeDtypeStruct((NC, NP), jnp.float32),
    mesh=_mesh(),
    scratch_types=[
        pltpu.VMEM((RPT, B), jnp.int32),
        pltpu.VMEM((112,), jnp.float32),
        pltpu.VMEM((TR,), jnp.float32),
        pltpu.VMEM_SHARED((NP,), jnp.float32),
    ],
)
def _deg_kernel(dst_hbm, out_hbm, idx_v, ones_v, zbuf_v, acc_sh):
    c = lax.axis_index("c")
    s = lax.axis_index("s")
    wid = s * NC + c

    def fill_ones(i, _):
        ones_v[pl.ds(i * 16, 16)] = jnp.full((16,), 1.0, jnp.float32)
        return 0

    lax.fori_loop(0, 112 // 16, fill_ones, 0)

    def fill_zero(i, _):
        zbuf_v[pl.ds(i * 16, 16)] = jnp.zeros((16,), jnp.float32)
        return 0

    lax.fori_loop(0, TR // 16, fill_zero, 0)

    pltpu.sync_copy(zbuf_v, acc_sh.at[pl.ds(s * TR, TR)])
    pltpu.sync_copy(dst_hbm.at[wid], idx_v)
    plsc.subcore_barrier()

    def body(j, _):
        pltpu.sync_copy(ones_v.at[pl.ds(0, B)], acc_sh.at[idx_v.at[j]],
                        add=True)
        return 0

    lax.fori_loop(0, RPT, body, 0)
    plsc.subcore_barrier()
    pltpu.sync_copy(acc_sh.at[pl.ds(s * TR, TR)],
                    out_hbm.at[c, pl.ds(s * TR, TR)])


@functools.partial(
    pl.kernel,
    out_type=jax.ShapeDtypeStruct((NC, NP, D), jnp.float32),
    mesh=_mesh(),
    scratch_types=[
        pltpu.VMEM((IB, B), jnp.int32),
        pltpu.VMEM((IB, B), jnp.int32),
        pltpu.VMEM((3, B, D), jnp.float32),
        pltpu.VMEM_SHARED((NP, D), jnp.float32),
        pltpu.SemaphoreType.DMA,
        pltpu.SemaphoreType.DMA,
    ],
)
def _edge_kernel(y_hbm, src_hbm, dst_hbm, out_hbm,
                 src_v, dst_v, rows_v, acc_sh, sem_g, sem_s):
    c = lax.axis_index("c")
    s = lax.axis_index("s")
    wid = s * NC + c

    def zrow(r, _):
        def zcol(k, _):
            rows_v[0, r, pl.ds(k * 16, 16)] = jnp.zeros((16,), jnp.float32)
            return 0

        lax.fori_loop(0, D // 16, zcol, 0)
        return 0

    lax.fori_loop(0, B, zrow, 0)

    for u in range(TR // 80):
        pltpu.sync_copy(rows_v.at[0, pl.ds(0, 80)],
                        acc_sh.at[pl.ds(s * TR + u * 80, 80)])

    plsc.subcore_barrier()

    def block(bk, _):
        pltpu.sync_copy(src_hbm.at[wid, bk], src_v)
        pltpu.sync_copy(dst_hbm.at[wid, bk], dst_v)
        pltpu.async_copy(y_hbm.at[src_v.at[0]], rows_v.at[0], sem_g)
        pltpu.async_copy(y_hbm.at[src_v.at[1]], rows_v.at[1], sem_g)

        def body(j, _):
            cur = lax.rem(j, 3)
            pltpu.make_async_copy(y_hbm.at[src_v.at[j]], rows_v.at[cur],
                                  sem_g).wait()

            @pl.when(j >= 1)
            def _():
                prev = lax.rem(j + 2, 3)
                pltpu.make_async_copy(rows_v.at[prev],
                                      acc_sh.at[dst_v.at[j - 1]],
                                      sem_s).wait()

            @pl.when(j + 2 < IB)
            def _():
                pltpu.async_copy(y_hbm.at[src_v.at[j + 2]],
                                 rows_v.at[lax.rem(j + 2, 3)], sem_g)

            pltpu.async_copy(rows_v.at[cur], acc_sh.at[dst_v.at[j]], sem_s,
                             add=True)
            return 0

        lax.fori_loop(0, IB, body, 0)
        pltpu.make_async_copy(rows_v.at[lax.rem(IB - 1, 3)],
                              acc_sh.at[dst_v.at[IB - 1]], sem_s).wait()
        return 0

    lax.fori_loop(0, NB, block, 0)
    plsc.subcore_barrier()
    pltpu.sync_copy(acc_sh.at[pl.ds(s * TR, TR)],
                    out_hbm.at[c, pl.ds(s * TR, TR)])


_RB = 1000
_G = N // _RB


def _rows_spec():
    return pl.BlockSpec((_RB, D), lambda i: (i, 0))


def _matmul_scale(x, w, dis):
    def body(x_ref, w_ref, d_ref, y_ref):
        y_ref[...] = jnp.dot(x_ref[...], w_ref[...],
                             preferred_element_type=jnp.float32) * d_ref[...]

    return pl.pallas_call(
        body,
        grid=(_G,),
        in_specs=[_rows_spec(),
                  pl.BlockSpec((D, D), lambda i: (0, 0)),
                  pl.BlockSpec((_RB, 1), lambda i: (i, 0))],
        out_specs=_rows_spec(),
        out_shape=jax.ShapeDtypeStruct((N, D), jnp.float32),
    )(x, w, dis)


def _part_spec(p):
    return pl.BlockSpec((1, _RB, D), lambda i, _p=p: (_p, i, 0))


def _mid_layer(sp, y1, dis, b1, w2):
    def body(s0_ref, s1_ref, y_ref, d_ref, b_ref, w_ref, o_ref):
        d = d_ref[...]
        h = jnp.maximum(d * (s0_ref[0] + s1_ref[0] + y_ref[...])
                        + b_ref[...], 0.0)
        o_ref[...] = jnp.dot(h, w_ref[...],
                             preferred_element_type=jnp.float32) * d

    return pl.pallas_call(
        body,
        grid=(_G,),
        in_specs=[_part_spec(0), _part_spec(1), _rows_spec(),
                  pl.BlockSpec((_RB, 1), lambda i: (i, 0)),
                  pl.BlockSpec((1, D), lambda i: (0, 0)),
                  pl.BlockSpec((D, D), lambda i: (0, 0))],
        out_specs=_rows_spec(),
        out_shape=jax.ShapeDtypeStruct((N, D), jnp.float32),
    )(sp, sp, y1, dis, b1, w2)


def _final_layer(sp, y2, dis, b2):
    def body(s0_ref, s1_ref, y_ref, d_ref, b_ref, o_ref):
        o_ref[...] = (d_ref[...] * (s0_ref[0] + s1_ref[0] + y_ref[...])
                      + b_ref[...])

    return pl.pallas_call(
        body,
        grid=(_G,),
        in_specs=[_part_spec(0), _part_spec(1), _rows_spec(),
                  pl.BlockSpec((_RB, 1), lambda i: (i, 0)),
                  pl.BlockSpec((1, D), lambda i: (0, 0))],
        out_specs=_rows_spec(),
        out_shape=jax.ShapeDtypeStruct((N, D), jnp.float32),
    )(sp, sp, y2, dis, b2)


def kernel(x, edge_index, W1, b1, W2, b2):
    ei = edge_index.astype(jnp.int32)
    src2 = ei[0].reshape(NW, NB, IB, B)
    dst2 = ei[1].reshape(NW, NB, IB, B)
    dstd = ei[1].reshape(NW, RPT, B)

    dp = _deg_kernel(dstd)
    deg = dp[0, :N] + dp[1, :N] + 1.0
    dis = lax.rsqrt(deg)[:, None]

    y1 = _matmul_scale(x, W1, dis)
    sp1 = _edge_kernel(y1, src2, dst2)
    y2 = _mid_layer(sp1, y1, dis, b1.reshape(1, D), W2)
    sp2 = _edge_kernel(y2, src2, dst2)
    return _final_layer(sp2, y2, dis, b2.reshape(1, D))
